# baseline (speedup 1.0000x reference)
import jax
import jax.numpy as jnp
from jax.experimental import pallas as pl
from jax.experimental.pallas import tpu as pltpu

B, D, H, K, L = 1024, 32, 64, 512, 3
B_TILE = 256
NB = B // B_TILE
TCH = 2048
NCH = (H * K) // TCH


def _tr_kernel(w_ref, lb_ref, wt_ref, lbt_ref):
    c = pl.program_id(1)
    wt_ref[0] = jnp.swapaxes(w_ref[0], 0, 1)

    @pl.when(c == 0)
    def _lb():
        lbt_ref[0] = jnp.swapaxes(lb_ref[0], 0, 1)


def _enc_kernel(x_ref, bw_ref, bb_ref, w_ref, lb_ref,
                enc_ref, out_ref, cur_ref, delta_ref):
    i = pl.program_id(0)
    b = pl.program_id(1)
    bs = pl.ds(b * B_TILE, B_TILE)

    @pl.when(i == 0)
    def _init():
        cur_ref[bs, :] = jnp.zeros((B_TILE, D), jnp.float32)

    cur = cur_ref[bs, :]
    u = jnp.dot(cur, bw_ref[...], preferred_element_type=jnp.float32)
    u = jnp.maximum(u + bb_ref[...], 0.0)

    def sq_d(d):
        mm = jnp.dot(u, w_ref[0, d], preferred_element_type=jnp.float32)
        ld = lb_ref[0, d] + mm
        delta_ref[d, :, :] = ld
        cd = cur[:, d:d + 1] + ld
        fd = cd - x_ref[bs, d:d + 1]
        return fd * fd

    level = [sq_d(d) + sq_d(d + 16) for d in range(16)]
    while len(level) > 1:
        half = len(level) // 2
        level = [level[j] + level[j + half] for j in range(half)]
    losses = level[0] * jnp.float32(1.0 / D)

    targ = jnp.argmin(losses, axis=-1).astype(jnp.int32)[:, None]
    li = jax.lax.broadcasted_iota(jnp.int32, (B_TILE, L), 1)
    enc_ref[bs, :] = jnp.where(li == i, targ, enc_ref[bs, :])
    mask = jax.lax.broadcasted_iota(jnp.int32, (B_TILE, K), 1) == targ
    cols = [jnp.sum(jnp.where(mask, delta_ref[d, :, :], 0.0),
                    axis=1, keepdims=True) for d in range(D)]
    newcur = cur + jnp.concatenate(cols, axis=1)
    cur_ref[bs, :] = newcur

    @pl.when(i == L - 1)
    def _done():
        out_ref[bs, :] = newcur


def kernel(inputs, base_W, base_b, layer_Ws, layer_biases):
    wt, lbt = pl.pallas_call(
        _tr_kernel,
        grid=(L, NCH),
        in_specs=[
            pl.BlockSpec((1, TCH, D), lambda i, c: (i, c, 0)),
            pl.BlockSpec((1, K, D), lambda i, c: (i, 0, 0)),
        ],
        out_specs=[
            pl.BlockSpec((1, D, TCH), lambda i, c: (i, 0, c)),
            pl.BlockSpec((1, D, K), lambda i, c: (i, 0, 0)),
        ],
        out_shape=[
            jax.ShapeDtypeStruct((L, D, H * K), jnp.float32),
            jax.ShapeDtypeStruct((L, D, K), jnp.float32),
        ],
    )(layer_Ws.reshape(L, H * K, D), layer_biases)
    wd = wt.reshape(L, D, H, K)
    lbd = lbt.reshape(L, D, 1, K)
    enc, cur = pl.pallas_call(
        _enc_kernel,
        grid=(L, NB),
        in_specs=[
            pl.BlockSpec((B, D), lambda i, b: (0, 0)),
            pl.BlockSpec((D, H), lambda i, b: (0, 0)),
            pl.BlockSpec((1, H), lambda i, b: (0, 0)),
            pl.BlockSpec((1, D, H, K), lambda i, b: (i, 0, 0, 0)),
            pl.BlockSpec((1, D, 1, K), lambda i, b: (i, 0, 0, 0)),
        ],
        out_specs=[
            pl.BlockSpec((B, L), lambda i, b: (0, 0)),
            pl.BlockSpec((B, D), lambda i, b: (0, 0)),
        ],
        out_shape=[
            jax.ShapeDtypeStruct((B, L), jnp.int32),
            jax.ShapeDtypeStruct((B, D), jnp.float32),
        ],
        scratch_shapes=[
            pltpu.VMEM((B, D), jnp.float32),
            pltpu.VMEM((D, B_TILE, K), jnp.float32),
        ],
    )(inputs, base_W, base_b.reshape(1, H), wd, lbd)
    return enc, cur


# speedup vs baseline: 1.6102x; 1.6102x over previous
import jax
import jax.numpy as jnp
from jax.experimental import pallas as pl
from jax.experimental.pallas import tpu as pltpu

B, D, H, K, L = 1024, 32, 64, 512, 3
B_TILE = 256
NB = B // B_TILE
def _tr_kernel(w_ref, lb_ref, wt_ref, lbt_ref):
    w = w_ref[0].reshape(H, K, D)
    wt_ref[0] = jnp.transpose(w, (2, 0, 1))
    lbt_ref[0] = jnp.swapaxes(lb_ref[0], 0, 1)


def _enc_kernel(x_ref, bw_ref, bb_ref, w_ref, lb_ref,
                enc_ref, out_ref, cur_ref, delta_ref):
    i = pl.program_id(0)
    b = pl.program_id(1)
    bs = pl.ds(b * B_TILE, B_TILE)

    @pl.when(i == 0)
    def _init():
        cur_ref[bs, :] = jnp.zeros((B_TILE, D), jnp.float32)

    cur = cur_ref[bs, :]
    u = jnp.dot(cur, bw_ref[...], preferred_element_type=jnp.float32)
    u = jnp.maximum(u + bb_ref[...], 0.0)

    def sq_d(d):
        mm = jnp.dot(u, w_ref[0, d], preferred_element_type=jnp.float32)
        ld = lb_ref[0, d] + mm
        delta_ref[d, :, :] = ld
        cd = cur[:, d:d + 1] + ld
        fd = cd - x_ref[bs, d:d + 1]
        return fd * fd

    level = [sq_d(d) + sq_d(d + 16) for d in range(16)]
    while len(level) > 1:
        half = len(level) // 2
        level = [level[j] + level[j + half] for j in range(half)]
    losses = level[0] * jnp.float32(1.0 / D)

    targ = jnp.argmin(losses, axis=-1).astype(jnp.int32)[:, None]
    li = jax.lax.broadcasted_iota(jnp.int32, (B_TILE, L), 1)
    enc_ref[bs, :] = jnp.where(li == i, targ, enc_ref[bs, :])
    mask = jax.lax.broadcasted_iota(jnp.int32, (B_TILE, K), 1) == targ
    cols = [jnp.sum(jnp.where(mask, delta_ref[d, :, :], 0.0),
                    axis=1, keepdims=True) for d in range(D)]
    newcur = cur + jnp.concatenate(cols, axis=1)
    cur_ref[bs, :] = newcur

    @pl.when(i == L - 1)
    def _done():
        out_ref[bs, :] = newcur


def kernel(inputs, base_W, base_b, layer_Ws, layer_biases):
    wd, lbt = pl.pallas_call(
        _tr_kernel,
        grid=(L,),
        in_specs=[
            pl.BlockSpec((1, H, K * D), lambda i: (i, 0, 0)),
            pl.BlockSpec((1, K, D), lambda i: (i, 0, 0)),
        ],
        out_specs=[
            pl.BlockSpec((1, D, H, K), lambda i: (i, 0, 0, 0)),
            pl.BlockSpec((1, D, K), lambda i: (i, 0, 0)),
        ],
        out_shape=[
            jax.ShapeDtypeStruct((L, D, H, K), jnp.float32),
            jax.ShapeDtypeStruct((L, D, K), jnp.float32),
        ],
    )(layer_Ws, layer_biases)
    lbd = lbt.reshape(L, D, 1, K)
    enc, cur = pl.pallas_call(
        _enc_kernel,
        grid=(L, NB),
        in_specs=[
            pl.BlockSpec((B, D), lambda i, b: (0, 0)),
            pl.BlockSpec((D, H), lambda i, b: (0, 0)),
            pl.BlockSpec((1, H), lambda i, b: (0, 0)),
            pl.BlockSpec((1, D, H, K), lambda i, b: (i, 0, 0, 0)),
            pl.BlockSpec((1, D, 1, K), lambda i, b: (i, 0, 0, 0)),
        ],
        out_specs=[
            pl.BlockSpec((B, L), lambda i, b: (0, 0)),
            pl.BlockSpec((B, D), lambda i, b: (0, 0)),
        ],
        out_shape=[
            jax.ShapeDtypeStruct((B, L), jnp.int32),
            jax.ShapeDtypeStruct((B, D), jnp.float32),
        ],
        scratch_shapes=[
            pltpu.VMEM((B, D), jnp.float32),
            pltpu.VMEM((D, B_TILE, K), jnp.float32),
        ],
    )(inputs, base_W, base_b.reshape(1, H), wd, lbd)
    return enc, cur


# speedup vs baseline: 1.6967x; 1.0537x over previous
import jax
import jax.numpy as jnp
from jax.experimental import pallas as pl
from jax.experimental.pallas import tpu as pltpu

B, D, H, K, L = 1024, 32, 64, 512, 3
B_TILE = 256
NB = B // B_TILE


def _enc_kernel(x_ref, bw_ref, bb_ref, w_ref, lb_ref,
                enc_ref, out_ref, cur_ref, delta_ref, wd_ref, lbd_ref):
    i = pl.program_id(0)
    b = pl.program_id(1)
    bs = pl.ds(b * B_TILE, B_TILE)

    @pl.when(i == 0)
    def _init():
        cur_ref[bs, :] = jnp.zeros((B_TILE, D), jnp.float32)

    @pl.when(b == 0)
    def _permute_weights():
        w = w_ref[0].reshape(H, K, D)
        wd_ref[...] = jnp.transpose(w, (2, 0, 1))
        lbd_ref[...] = jnp.swapaxes(lb_ref[0], 0, 1)

    cur = cur_ref[bs, :]
    u = jnp.dot(cur, bw_ref[...], preferred_element_type=jnp.float32)
    u = jnp.maximum(u + bb_ref[...], 0.0)

    def sq_d(d):
        mm = jnp.dot(u, wd_ref[d], preferred_element_type=jnp.float32)
        ld = lbd_ref[d:d + 1, :] + mm
        delta_ref[d, :, :] = ld
        cd = cur[:, d:d + 1] + ld
        fd = cd - x_ref[bs, d:d + 1]
        return fd * fd

    level = [sq_d(d) + sq_d(d + 16) for d in range(16)]
    while len(level) > 1:
        half = len(level) // 2
        level = [level[j] + level[j + half] for j in range(half)]
    losses = level[0] * jnp.float32(1.0 / D)

    targ = jnp.argmin(losses, axis=-1).astype(jnp.int32)[:, None]
    li = jax.lax.broadcasted_iota(jnp.int32, (B_TILE, L), 1)
    enc_ref[bs, :] = jnp.where(li == i, targ, enc_ref[bs, :])
    mask = jax.lax.broadcasted_iota(jnp.int32, (B_TILE, K), 1) == targ
    cols = [jnp.sum(jnp.where(mask, delta_ref[d, :, :], 0.0),
                    axis=1, keepdims=True) for d in range(D)]
    newcur = cur + jnp.concatenate(cols, axis=1)
    cur_ref[bs, :] = newcur

    @pl.when(i == L - 1)
    def _done():
        out_ref[bs, :] = newcur


def kernel(inputs, base_W, base_b, layer_Ws, layer_biases):
    enc, cur = pl.pallas_call(
        _enc_kernel,
        grid=(L, NB),
        in_specs=[
            pl.BlockSpec((B, D), lambda i, b: (0, 0)),
            pl.BlockSpec((D, H), lambda i, b: (0, 0)),
            pl.BlockSpec((1, H), lambda i, b: (0, 0)),
            pl.BlockSpec((1, H, K * D), lambda i, b: (i, 0, 0)),
            pl.BlockSpec((1, K, D), lambda i, b: (i, 0, 0)),
        ],
        out_specs=[
            pl.BlockSpec((B, L), lambda i, b: (0, 0)),
            pl.BlockSpec((B, D), lambda i, b: (0, 0)),
        ],
        out_shape=[
            jax.ShapeDtypeStruct((B, L), jnp.int32),
            jax.ShapeDtypeStruct((B, D), jnp.float32),
        ],
        scratch_shapes=[
            pltpu.VMEM((B, D), jnp.float32),
            pltpu.VMEM((D, B_TILE, K), jnp.float32),
            pltpu.VMEM((D, H, K), jnp.float32),
            pltpu.VMEM((D, K), jnp.float32),
        ],
    )(inputs, base_W, base_b.reshape(1, H), layer_Ws, layer_biases)
    return enc, cur


# speedup vs baseline: 1.7330x; 1.0214x over previous
import jax
import jax.numpy as jnp
from jax.experimental import pallas as pl
from jax.experimental.pallas import tpu as pltpu

B, D, H, K, L = 1024, 32, 64, 512, 3
B_TILE = 256
NB = B // B_TILE


def _enc_kernel(x_ref, bw_ref, bb_ref, w_ref, lb_ref,
                enc_ref, out_ref, cur_ref, delta_ref, wd_ref, lbd_ref):
    i = pl.program_id(0)
    b = pl.program_id(1)
    bs = pl.ds(b * B_TILE, B_TILE)

    @pl.when(i == 0)
    def _init():
        cur_ref[bs, :] = jnp.zeros((B_TILE, D), jnp.float32)

    @pl.when(b == 0)
    def _permute_weights():
        w = w_ref[0].reshape(H, K, D)
        wd_ref[...] = jnp.transpose(w, (2, 0, 1))
        lbd_ref[...] = jnp.swapaxes(lb_ref[0], 0, 1)

    cur = cur_ref[bs, :]
    u = jnp.dot(cur, bw_ref[...], preferred_element_type=jnp.float32)
    u = jnp.maximum(u + bb_ref[...], 0.0)

    def sq_d(d):
        mm = jnp.dot(u, wd_ref[d], preferred_element_type=jnp.float32)
        ld = lbd_ref[d:d + 1, :] + mm
        delta_ref[d, :, :] = ld
        cd = cur[:, d:d + 1] + ld
        fd = cd - x_ref[bs, d:d + 1]
        return fd * fd

    def tree(n, j):
        if n == 1:
            return sq_d(j) + sq_d(j + 16)
        return tree(n - 1, j) + tree(n - 1, j + (16 >> (n - 1)))

    losses = tree(5, 0) * jnp.float32(1.0 / D)

    targ = jnp.argmin(losses, axis=-1).astype(jnp.int32)[:, None]
    li = jax.lax.broadcasted_iota(jnp.int32, (B_TILE, L), 1)
    enc_ref[bs, :] = jnp.where(li == i, targ, enc_ref[bs, :])
    mask = jax.lax.broadcasted_iota(jnp.int32, (B_TILE, K), 1) == targ
    cols = [jnp.sum(jnp.where(mask, delta_ref[d, :, :], 0.0),
                    axis=1, keepdims=True) for d in range(D)]
    newcur = cur + jnp.concatenate(cols, axis=1)
    cur_ref[bs, :] = newcur

    @pl.when(i == L - 1)
    def _done():
        out_ref[bs, :] = newcur


def kernel(inputs, base_W, base_b, layer_Ws, layer_biases):
    enc, cur = pl.pallas_call(
        _enc_kernel,
        grid=(L, NB),
        in_specs=[
            pl.BlockSpec((B, D), lambda i, b: (0, 0)),
            pl.BlockSpec((D, H), lambda i, b: (0, 0)),
            pl.BlockSpec((1, H), lambda i, b: (0, 0)),
            pl.BlockSpec((1, H, K * D), lambda i, b: (i, 0, 0)),
            pl.BlockSpec((1, K, D), lambda i, b: (i, 0, 0)),
        ],
        out_specs=[
            pl.BlockSpec((B, L), lambda i, b: (0, 0)),
            pl.BlockSpec((B, D), lambda i, b: (0, 0)),
        ],
        out_shape=[
            jax.ShapeDtypeStruct((B, L), jnp.int32),
            jax.ShapeDtypeStruct((B, D), jnp.float32),
        ],
        scratch_shapes=[
            pltpu.VMEM((B, D), jnp.float32),
            pltpu.VMEM((D, B_TILE, K), jnp.float32),
            pltpu.VMEM((D, H, K), jnp.float32),
            pltpu.VMEM((D, K), jnp.float32),
        ],
    )(inputs, base_W, base_b.reshape(1, H), layer_Ws, layer_biases)
    return enc, cur
